# Initial kernel scaffold; baseline (speedup 1.0000x reference)
import jax
import jax.numpy as jnp
from jax.experimental import pallas as pl


def kernel(x):
    raise NotImplementedError("write your pallas kernel here")


# speedup vs baseline: 3.0126x; 3.0126x over previous
import jax
import jax.numpy as jnp
from jax.experimental import pallas as pl
from jax.experimental.pallas import tpu as pltpu

_BC = 512


def _scan_body(x_ref, o_ref):
    x = x_ref[0]
    n, bc = x.shape
    k = 1
    while k < n:
        shifted = jnp.concatenate(
            [jnp.zeros((k, bc), x.dtype), x[: n - k, :]], axis=0
        )
        x = x + shifted
        k *= 2
    o_ref[0] = x


def kernel(x):
    b, n, m = x.shape
    grid = (b, m // _BC)
    return pl.pallas_call(
        _scan_body,
        grid=grid,
        in_specs=[pl.BlockSpec((1, n, _BC), lambda i, j: (i, 0, j))],
        out_specs=pl.BlockSpec((1, n, _BC), lambda i, j: (i, 0, j)),
        out_shape=jax.ShapeDtypeStruct((b, n, m), x.dtype),
        compiler_params=pltpu.CompilerParams(
            dimension_semantics=("parallel", "parallel"),
        ),
    )(x)


# speedup vs baseline: 3.5819x; 1.1890x over previous
import jax
import jax.numpy as jnp
from jax.experimental import pallas as pl
from jax.experimental.pallas import tpu as pltpu

_BC = 512
_C = 256


def _scan_body(x_ref, o_ref):
    x = x_ref[0]
    n, bc = x.shape
    ii = jax.lax.broadcasted_iota(jnp.int32, (_C, _C), 0)
    jj = jax.lax.broadcasted_iota(jnp.int32, (_C, _C), 1)
    tri = (jj <= ii).astype(jnp.bfloat16)
    carry = jnp.zeros((bc,), jnp.float32)
    for i in range(n // _C):
        xi = x[i * _C : (i + 1) * _C, :]
        hi = xi.astype(jnp.bfloat16)
        lo = (xi - hi.astype(jnp.float32)).astype(jnp.bfloat16)
        yi = jax.lax.dot(
            tri, hi, preferred_element_type=jnp.float32
        ) + jax.lax.dot(tri, lo, preferred_element_type=jnp.float32)
        yi = yi + carry
        carry = yi[_C - 1]
        o_ref[0, i * _C : (i + 1) * _C, :] = yi


def kernel(x):
    b, n, m = x.shape
    grid = (b, m // _BC)
    return pl.pallas_call(
        _scan_body,
        grid=grid,
        in_specs=[pl.BlockSpec((1, n, _BC), lambda i, j: (i, 0, j))],
        out_specs=pl.BlockSpec((1, n, _BC), lambda i, j: (i, 0, j)),
        out_shape=jax.ShapeDtypeStruct((b, n, m), x.dtype),
        compiler_params=pltpu.CompilerParams(
            dimension_semantics=("parallel", "parallel"),
        ),
    )(x)
